# baseline (speedup 1.0000x reference)
import functools

import jax
import jax.numpy as jnp
from jax import lax
from jax.experimental import pallas as pl
from jax.experimental.pallas import tpu as pltpu

BATCH = 4096
EXP_DIM = 8192
NSEL = 32
CHUNK = 256
POOL = 1024.0

ROW_BLK = 256


def _tc_body(x_ref, o_ref):
    pos = lax.broadcasted_iota(jnp.int32, (1, CHUNK), 1).astype(jnp.float32)
    w = jnp.mod(pos * 2654435761.0, 1000000007.0)
    for i in range(NSEL):
        chunk = x_ref[:, i * CHUNK:(i + 1) * CHUNK]
        h = jnp.sum(chunk * w, axis=1)
        o_ref[:, i] = jnp.mod(h, POOL).astype(jnp.int32)


def kernel(sparse_code):
    B, D = sparse_code.shape
    grid = (B // ROW_BLK,)
    return pl.pallas_call(
        _tc_body,
        grid=grid,
        in_specs=[pl.BlockSpec((ROW_BLK, D), lambda i: (i, 0))],
        out_specs=pl.BlockSpec((ROW_BLK, NSEL), lambda i: (i, 0)),
        out_shape=jax.ShapeDtypeStruct((B, NSEL), jnp.int32),
    )(sparse_code)
